# baseline (speedup 1.0000x reference)
import functools

import numpy as np

import jax
import jax.numpy as jnp
from jax import lax
from jax.experimental import pallas as pl
from jax.experimental.pallas import tpu as pltpu
from jax.experimental.pallas import tpu_sc as plsc

B = 16384
D = 128
UF = 64
CD = 128
H = 20
NC = 2
NS = 16
NW = NC * NS
S = B // NW
CHUNK = 128
N_IDC = S // CHUNK
N_TAG = S * H // CHUNK
NBUF = 2


def _sc_gather_body(tags3, uids3, iids3, pos3, user_table, tag_table,
                    item_table, uid_out, tsum_out, iid_out,
                    tidx_v, pos_v, idx_v, rows0, rows1, acc_sh,
                    sem0, sem1):
    cid = lax.axis_index("c")
    sid = lax.axis_index("s")
    wid = sid * NC + cid
    base = wid * S
    slab = sid * S
    bufs = (rows0, rows1)
    sems = (sem0, sem1)

    def _zero_row(i, carry):
        for cc in range(D // 16):
            rows0[i, pl.ds(cc * 16, 16)] = jnp.zeros((16,), jnp.float32)
        return carry
    lax.fori_loop(0, CHUNK, _zero_row, 0)
    for c in range(S // CHUNK):
        pltpu.sync_copy(rows0, acc_sh.at[pl.ds(slab + c * CHUNK, CHUNK)])

    for ids3, table, out in ((uids3, user_table, uid_out),
                             (iids3, item_table, iid_out)):
        pltpu.sync_copy(ids3.at[wid], idx_v)
        cps = [pltpu.async_copy(table.at[idx_v.at[0]], bufs[0], sems[0])]
        for c in range(N_IDC):
            if c + 1 < N_IDC:
                cps.append(pltpu.async_copy(table.at[idx_v.at[c + 1]],
                                            bufs[(c + 1) % 2],
                                            sems[(c + 1) % 2]))
            cps[c].wait()
            pltpu.sync_copy(bufs[c % 2],
                            out.at[pl.ds(base + c * CHUNK, CHUNK)])

    pltpu.sync_copy(tags3.at[wid], tidx_v)
    pltpu.sync_copy(pos3.at[sid], pos_v)

    for k in range(NBUF - 1):
        pltpu.async_copy(tag_table.at[tidx_v.at[k]], bufs[k], sems[k])

    def _tag_quad(i, carry):
        for k in range(NBUF):
            c = NBUF * i + k

            @pl.when(c + NBUF - 1 < N_TAG)
            def _():
                pltpu.async_copy(tag_table.at[tidx_v.at[c + NBUF - 1]],
                                 bufs[(k + NBUF - 1) % NBUF],
                                 sems[(k + NBUF - 1) % NBUF])

            pltpu.make_async_copy(tag_table.at[tidx_v.at[c]], bufs[k],
                                  sems[k]).wait()
            pltpu.sync_copy(bufs[k], acc_sh.at[pos_v.at[c]], add=True)
        return carry
    lax.fori_loop(0, N_TAG // NBUF, _tag_quad, 0)

    pltpu.sync_copy(acc_sh.at[pl.ds(slab, S)], tsum_out.at[pl.ds(base, S)])


def _sc_gather(user_ids, user_tags_idx, item_ids, user_table, tag_table,
               item_table):
    tags3 = user_tags_idx.reshape(NW, N_TAG, CHUNK)
    uids3 = user_ids.reshape(NW, N_IDC, CHUNK)
    iids3 = item_ids.reshape(NW, N_IDC, CHUNK)
    pos3 = jnp.asarray((np.arange(NS, dtype=np.int32)[:, None] * S
                        + np.repeat(np.arange(S, dtype=np.int32), H)[None, :]
                        ).reshape(NS, N_TAG, CHUNK))
    mesh = plsc.VectorSubcoreMesh(core_axis_name="c", subcore_axis_name="s")
    fn = pl.kernel(
        _sc_gather_body,
        out_type=[jax.ShapeDtypeStruct((B, D), jnp.float32) for _ in range(3)],
        mesh=mesh,
        scratch_types=[
            pltpu.VMEM((N_TAG, CHUNK), jnp.int32),
            pltpu.VMEM((N_TAG, CHUNK), jnp.int32),
            pltpu.VMEM((N_IDC, CHUNK), jnp.int32),
            pltpu.VMEM((CHUNK, D), jnp.float32),
            pltpu.VMEM((CHUNK, D), jnp.float32),
            pltpu.VMEM_SHARED((NS * S, D), jnp.float32),
            pltpu.SemaphoreType.DMA,
            pltpu.SemaphoreType.DMA,
        ],
    )
    return fn(tags3, uids3, iids3, pos3, user_table, tag_table, item_table)


def _dot(a, w):
    return lax.dot_general(a.astype(jnp.bfloat16), w.astype(jnp.bfloat16),
                           (((1,), (0,)), ((), ())),
                           preferred_element_type=jnp.float32)


def _ln_aff(x, g, b, eps=1e-5):
    m = jnp.mean(x, axis=-1, keepdims=True)
    v = jnp.mean((x - m) ** 2, axis=-1, keepdims=True)
    return (x - m) * lax.rsqrt(v + eps) * g + b


def _tc_body(uid, tsum, iid, cf, tp_W, tp_b, tp_g, tp_beta, uf_W, uf_b, uf_g,
             uf_beta, cf_W, cf_b, cf_g, cf_beta, if_W, if_b, if_g, if_beta,
             m1_W, m1_b, m2_W, m2_b, m3_W, m3_b, p_W, p_b, out_ref):
    tag = tsum[...] * (1.0 / H)
    t = _ln_aff(jax.nn.relu(_dot(tag, tp_W[...]) + tp_b[...]),
                tp_g[...], tp_beta[...])
    ufW = uf_W[...]
    ue = _ln_aff(jax.nn.relu(_dot(uid[...], ufW[:D]) + _dot(t, ufW[D:])
                             + uf_b[...]), uf_g[...], uf_beta[...])
    ce = _ln_aff(jax.nn.relu(_dot(cf[...], cf_W[...]) + cf_b[...]),
                 cf_g[...], cf_beta[...])
    ifW = if_W[...]
    ie = _ln_aff(jax.nn.relu(_dot(iid[...], ifW[:D]) + _dot(ce, ifW[D:])
                             + if_b[...]), if_g[...], if_beta[...])
    m1W = m1_W[...]
    h = jax.nn.relu(_dot(ue, m1W[:UF]) + _dot(ie, m1W[UF:]) + m1_b[...])
    h = jax.nn.relu(_dot(h, m2_W[...]) + m2_b[...])
    h = jax.nn.relu(_dot(h, m3_W[...]) + m3_b[...])
    logit = jnp.sum(h * p_W[...], axis=1, keepdims=True) + p_b[0, 0]
    out_ref[...] = jax.nn.sigmoid(logit)


def kernel(user_ids, user_tags_idx, item_ids, content_features, user_table,
           tag_table, item_table, tp_W, tp_b, tp_g, tp_beta, uf_W, uf_b, uf_g,
           uf_beta, cf_W, cf_b, cf_g, cf_beta, if_W, if_b, if_g, if_beta,
           m1_W, m1_b, m2_W, m2_b, m3_W, m3_b, p_W, p_b):
    uid, tsum, iid = _sc_gather(user_ids, user_tags_idx, item_ids,
                                user_table, tag_table, item_table)

    BM = 2048
    grid = (B // BM,)
    row = pl.BlockSpec((BM, D), lambda i: (i, 0))
    full = lambda a: pl.BlockSpec(a.shape, lambda i: tuple(0 for _ in a.shape))
    weights = [tp_W.T, tp_b.reshape(1, -1), tp_g.reshape(1, -1),
               tp_beta.reshape(1, -1), uf_W.T, uf_b.reshape(1, -1),
               uf_g.reshape(1, -1), uf_beta.reshape(1, -1), cf_W.T,
               cf_b.reshape(1, -1), cf_g.reshape(1, -1),
               cf_beta.reshape(1, -1), if_W.T, if_b.reshape(1, -1),
               if_g.reshape(1, -1), if_beta.reshape(1, -1), m1_W.T,
               m1_b.reshape(1, -1), m2_W.T, m2_b.reshape(1, -1), m3_W.T,
               m3_b.reshape(1, -1), p_W, p_b.reshape(1, -1)]
    out = pl.pallas_call(
        _tc_body,
        grid=grid,
        in_specs=[row, row, row, row] + [full(w) for w in weights],
        out_specs=pl.BlockSpec((BM, 1), lambda i: (i, 0)),
        out_shape=jax.ShapeDtypeStruct((B, 1), jnp.float32),
    )(uid, tsum, iid, content_features, *weights)
    return out.reshape(B)


# speedup vs baseline: 1.0033x; 1.0033x over previous
import functools

import numpy as np

import jax
import jax.numpy as jnp
from jax import lax
from jax.experimental import pallas as pl
from jax.experimental.pallas import tpu as pltpu
from jax.experimental.pallas import tpu_sc as plsc

B = 16384
D = 128
UF = 64
CD = 128
H = 20
NC = 2
NS = 16
NW = NC * NS
S = B // NW
CHUNK = 128
N_IDC = S // CHUNK
N_TAG = S * H // CHUNK
NBUF = 2


def _sc_gather_body(tags3, uids3, iids3, pos3, user_table, tag_table,
                    item_table, uid_out, tsum_out, iid_out,
                    tidx_v, pos_v, idx_v, rows0, rows1, acc_sh,
                    sem0, sem1):
    cid = lax.axis_index("c")
    sid = lax.axis_index("s")
    wid = sid * NC + cid
    base = wid * S
    slab = sid * S
    bufs = (rows0, rows1)
    sems = (sem0, sem1)

    def _zero_row(i, carry):
        for cc in range(D // 16):
            rows0[i, pl.ds(cc * 16, 16)] = jnp.zeros((16,), jnp.float32)
        return carry
    lax.fori_loop(0, CHUNK, _zero_row, 0)
    for c in range(S // CHUNK):
        pltpu.sync_copy(rows0, acc_sh.at[pl.ds(slab + c * CHUNK, CHUNK)])

    for ids3, table, out in ((uids3, user_table, uid_out),
                             (iids3, item_table, iid_out)):
        pltpu.sync_copy(ids3.at[wid], idx_v)
        cps = [pltpu.async_copy(table.at[idx_v.at[0]], bufs[0], sems[0])]
        for c in range(N_IDC):
            if c + 1 < N_IDC:
                cps.append(pltpu.async_copy(table.at[idx_v.at[c + 1]],
                                            bufs[(c + 1) % 2],
                                            sems[(c + 1) % 2]))
            cps[c].wait()
            pltpu.sync_copy(bufs[c % 2],
                            out.at[pl.ds(base + c * CHUNK, CHUNK)])

    pltpu.sync_copy(tags3.at[wid], tidx_v)
    pltpu.sync_copy(pos3.at[sid], pos_v)

    for k in range(NBUF - 1):
        pltpu.async_copy(tag_table.at[tidx_v.at[k]], bufs[k], sems[k])

    def _tag_quad(i, carry):
        for k in range(NBUF):
            c = NBUF * i + k

            @pl.when(c + NBUF - 1 < N_TAG)
            def _():
                pltpu.async_copy(tag_table.at[tidx_v.at[c + NBUF - 1]],
                                 bufs[(k + NBUF - 1) % NBUF],
                                 sems[(k + NBUF - 1) % NBUF])

            pltpu.make_async_copy(tag_table.at[tidx_v.at[c]], bufs[k],
                                  sems[k]).wait()
            pltpu.sync_copy(bufs[k], acc_sh.at[pos_v.at[c]], add=True)
        return carry
    lax.fori_loop(0, N_TAG // NBUF, _tag_quad, 0)

    pltpu.sync_copy(acc_sh.at[pl.ds(slab, S)], tsum_out.at[pl.ds(base, S)])


def _sc_gather(user_ids, user_tags_idx, item_ids, user_table, tag_table,
               item_table):
    tags3 = user_tags_idx.reshape(NW, N_TAG, CHUNK)
    uids3 = user_ids.reshape(NW, N_IDC, CHUNK)
    iids3 = item_ids.reshape(NW, N_IDC, CHUNK)
    pos3 = jnp.asarray((np.arange(NS, dtype=np.int32)[:, None] * S
                        + np.repeat(np.arange(S, dtype=np.int32), H)[None, :]
                        ).reshape(NS, N_TAG, CHUNK))
    mesh = plsc.VectorSubcoreMesh(core_axis_name="c", subcore_axis_name="s")
    fn = pl.kernel(
        _sc_gather_body,
        out_type=[jax.ShapeDtypeStruct((B, D), jnp.float32) for _ in range(3)],
        mesh=mesh,
        scratch_types=[
            pltpu.VMEM((N_TAG, CHUNK), jnp.int32),
            pltpu.VMEM((N_TAG, CHUNK), jnp.int32),
            pltpu.VMEM((N_IDC, CHUNK), jnp.int32),
            pltpu.VMEM((CHUNK, D), jnp.float32),
            pltpu.VMEM((CHUNK, D), jnp.float32),
            pltpu.VMEM_SHARED((NS * S, D), jnp.float32),
            pltpu.SemaphoreType.DMA,
            pltpu.SemaphoreType.DMA,
        ],
    )
    return fn(tags3, uids3, iids3, pos3, user_table, tag_table, item_table)


def _dot(a, w):
    return lax.dot_general(a.astype(jnp.bfloat16), w.astype(jnp.bfloat16),
                           (((1,), (0,)), ((), ())),
                           preferred_element_type=jnp.float32)


def _ln_aff(x, g, b, eps=1e-5):
    m = jnp.mean(x, axis=-1, keepdims=True)
    v = jnp.mean((x - m) ** 2, axis=-1, keepdims=True)
    return (x - m) * lax.rsqrt(v + eps) * g + b


def _tc_body(uid, tsum, iid, cf, tp_W, tp_b, tp_g, tp_beta, uf_W, uf_b, uf_g,
             uf_beta, cf_W, cf_b, cf_g, cf_beta, if_W, if_b, if_g, if_beta,
             m1_W, m1_b, m2_W, m2_b, m3_W, m3_b, p_W, p_b, out_ref):
    tag = tsum[...] * (1.0 / H)
    t = _ln_aff(jax.nn.relu(_dot(tag, tp_W[...]) + tp_b[...]),
                tp_g[...], tp_beta[...])
    ufW = uf_W[...]
    ue = _ln_aff(jax.nn.relu(_dot(uid[...], ufW[:D]) + _dot(t, ufW[D:])
                             + uf_b[...]), uf_g[...], uf_beta[...])
    ce = _ln_aff(jax.nn.relu(_dot(cf[...], cf_W[...]) + cf_b[...]),
                 cf_g[...], cf_beta[...])
    ifW = if_W[...]
    ie = _ln_aff(jax.nn.relu(_dot(iid[...], ifW[:D]) + _dot(ce, ifW[D:])
                             + if_b[...]), if_g[...], if_beta[...])
    m1W = m1_W[...]
    h = jax.nn.relu(_dot(ue, m1W[:UF]) + _dot(ie, m1W[UF:]) + m1_b[...])
    h = jax.nn.relu(_dot(h, m2_W[...]) + m2_b[...])
    h = jax.nn.relu(_dot(h, m3_W[...]) + m3_b[...])
    logit = jnp.sum(h * p_W[...], axis=1) + p_b[0, 0]
    out_ref[...] = jax.nn.sigmoid(logit)


def kernel(user_ids, user_tags_idx, item_ids, content_features, user_table,
           tag_table, item_table, tp_W, tp_b, tp_g, tp_beta, uf_W, uf_b, uf_g,
           uf_beta, cf_W, cf_b, cf_g, cf_beta, if_W, if_b, if_g, if_beta,
           m1_W, m1_b, m2_W, m2_b, m3_W, m3_b, p_W, p_b):
    uid, tsum, iid = _sc_gather(user_ids, user_tags_idx, item_ids,
                                user_table, tag_table, item_table)

    BM = 4096
    grid = (B // BM,)
    row = pl.BlockSpec((BM, D), lambda i: (i, 0))
    full = lambda a: pl.BlockSpec(a.shape, lambda i: tuple(0 for _ in a.shape))
    weights = [tp_W.T, tp_b.reshape(1, -1), tp_g.reshape(1, -1),
               tp_beta.reshape(1, -1), uf_W.T, uf_b.reshape(1, -1),
               uf_g.reshape(1, -1), uf_beta.reshape(1, -1), cf_W.T,
               cf_b.reshape(1, -1), cf_g.reshape(1, -1),
               cf_beta.reshape(1, -1), if_W.T, if_b.reshape(1, -1),
               if_g.reshape(1, -1), if_beta.reshape(1, -1), m1_W.T,
               m1_b.reshape(1, -1), m2_W.T, m2_b.reshape(1, -1), m3_W.T,
               m3_b.reshape(1, -1), p_W, p_b.reshape(1, -1)]
    out = pl.pallas_call(
        _tc_body,
        grid=grid,
        in_specs=[row, row, row, row] + [full(w) for w in weights],
        out_specs=pl.BlockSpec((BM,), lambda i: (i,)),
        out_shape=jax.ShapeDtypeStruct((B,), jnp.float32),
    )(uid, tsum, iid, content_features, *weights)
    return out


# speedup vs baseline: 1.0713x; 1.0677x over previous
import functools

import numpy as np

import jax
import jax.numpy as jnp
from jax import lax
from jax.experimental import pallas as pl
from jax.experimental.pallas import tpu as pltpu
from jax.experimental.pallas import tpu_sc as plsc

B = 16384
D = 128
UF = 64
CD = 128
H = 20
NC = 2
NS = 16
NW = NC * NS
P = 2
CHUNK = 128
NBUF = 4


def _sc_gather_body(s, n_idc, n_tag,
                    tags3, uids3, iids3, pos3, user_table, tag_table,
                    item_table, uid_out, tsum_out, iid_out,
                    tidx_v, pos_v, idx_v, rows0, rows1, rows2, rows3, acc_sh,
                    sem0, sem1, sem2, sem3):
    cid = lax.axis_index("c")
    sid = lax.axis_index("s")
    wid = sid * NC + cid
    base = wid * s
    slab = sid * s
    bufs = (rows0, rows1, rows2, rows3)
    sems = (sem0, sem1, sem2, sem3)

    def _zero_row(i, carry):
        for cc in range(D // 16):
            rows0[i, pl.ds(cc * 16, 16)] = jnp.zeros((16,), jnp.float32)
        return carry
    lax.fori_loop(0, CHUNK, _zero_row, 0)
    for c in range(s // CHUNK):
        pltpu.sync_copy(rows0, acc_sh.at[pl.ds(slab + c * CHUNK, CHUNK)])

    for ids3, table, out in ((uids3, user_table, uid_out),
                             (iids3, item_table, iid_out)):
        pltpu.sync_copy(ids3.at[wid], idx_v)
        cps = [pltpu.async_copy(table.at[idx_v.at[0]], bufs[0], sems[0])]
        for c in range(n_idc):
            if c + 1 < n_idc:
                cps.append(pltpu.async_copy(table.at[idx_v.at[c + 1]],
                                            bufs[(c + 1) % 2],
                                            sems[(c + 1) % 2]))
            cps[c].wait()
            pltpu.sync_copy(bufs[c % 2],
                            out.at[pl.ds(base + c * CHUNK, CHUNK)])

    pltpu.sync_copy(tags3.at[wid], tidx_v)
    pltpu.sync_copy(pos3.at[sid], pos_v)

    for k in range(NBUF - 1):
        pltpu.async_copy(tag_table.at[tidx_v.at[k]], bufs[k], sems[k])

    def _tag_quad(i, carry):
        for k in range(NBUF):
            c = NBUF * i + k

            @pl.when(c + NBUF - 1 < n_tag)
            def _():
                pltpu.async_copy(tag_table.at[tidx_v.at[c + NBUF - 1]],
                                 bufs[(k + NBUF - 1) % NBUF],
                                 sems[(k + NBUF - 1) % NBUF])

            pltpu.make_async_copy(tag_table.at[tidx_v.at[c]], bufs[k],
                                  sems[k]).wait()
            pltpu.sync_copy(bufs[k], acc_sh.at[pos_v.at[c]], add=True)
        return carry
    lax.fori_loop(0, n_tag // NBUF, _tag_quad, 0)

    pltpu.sync_copy(acc_sh.at[pl.ds(slab, s)], tsum_out.at[pl.ds(base, s)])


def _sc_gather(user_ids, user_tags_idx, item_ids, user_table, tag_table,
               item_table):
    n = user_ids.shape[0]
    s = n // NW
    n_idc = s // CHUNK
    n_tag = s * H // CHUNK
    tags3 = user_tags_idx.reshape(NW, n_tag, CHUNK)
    uids3 = user_ids.reshape(NW, n_idc, CHUNK)
    iids3 = item_ids.reshape(NW, n_idc, CHUNK)
    pos3 = jnp.asarray((np.arange(NS, dtype=np.int32)[:, None] * s
                        + np.repeat(np.arange(s, dtype=np.int32), H)[None, :]
                        ).reshape(NS, n_tag, CHUNK))
    mesh = plsc.VectorSubcoreMesh(core_axis_name="c", subcore_axis_name="s")
    fn = pl.kernel(
        functools.partial(_sc_gather_body, s, n_idc, n_tag),
        out_type=[jax.ShapeDtypeStruct((n, D), jnp.float32) for _ in range(3)],
        mesh=mesh,
        scratch_types=[
            pltpu.VMEM((n_tag, CHUNK), jnp.int32),
            pltpu.VMEM((n_tag, CHUNK), jnp.int32),
            pltpu.VMEM((n_idc, CHUNK), jnp.int32),
            pltpu.VMEM((CHUNK, D), jnp.float32),
            pltpu.VMEM((CHUNK, D), jnp.float32),
            pltpu.VMEM((CHUNK, D), jnp.float32),
            pltpu.VMEM((CHUNK, D), jnp.float32),
            pltpu.VMEM_SHARED((NS * s, D), jnp.float32),
            pltpu.SemaphoreType.DMA,
            pltpu.SemaphoreType.DMA,
            pltpu.SemaphoreType.DMA,
            pltpu.SemaphoreType.DMA,
        ],
    )
    return fn(tags3, uids3, iids3, pos3, user_table, tag_table, item_table)


def _dot(a, w):
    return lax.dot_general(a.astype(jnp.bfloat16), w.astype(jnp.bfloat16),
                           (((1,), (0,)), ((), ())),
                           preferred_element_type=jnp.float32)


def _ln_aff(x, g, b, eps=1e-5):
    m = jnp.mean(x, axis=-1, keepdims=True)
    v = jnp.mean((x - m) ** 2, axis=-1, keepdims=True)
    return (x - m) * lax.rsqrt(v + eps) * g + b


def _tc_body(uid, tsum, iid, cf, tp_W, tp_b, tp_g, tp_beta, uf_W, uf_b, uf_g,
             uf_beta, cf_W, cf_b, cf_g, cf_beta, if_W, if_b, if_g, if_beta,
             m1_W, m1_b, m2_W, m2_b, m3_W, m3_b, p_W, p_b, out_ref):
    tag = tsum[...] * (1.0 / H)
    t = _ln_aff(jax.nn.relu(_dot(tag, tp_W[...]) + tp_b[...]),
                tp_g[...], tp_beta[...])
    ufW = uf_W[...]
    ue = _ln_aff(jax.nn.relu(_dot(uid[...], ufW[:D]) + _dot(t, ufW[D:])
                             + uf_b[...]), uf_g[...], uf_beta[...])
    ce = _ln_aff(jax.nn.relu(_dot(cf[...], cf_W[...]) + cf_b[...]),
                 cf_g[...], cf_beta[...])
    ifW = if_W[...]
    ie = _ln_aff(jax.nn.relu(_dot(iid[...], ifW[:D]) + _dot(ce, ifW[D:])
                             + if_b[...]), if_g[...], if_beta[...])
    m1W = m1_W[...]
    h = jax.nn.relu(_dot(ue, m1W[:UF]) + _dot(ie, m1W[UF:]) + m1_b[...])
    h = jax.nn.relu(_dot(h, m2_W[...]) + m2_b[...])
    h = jax.nn.relu(_dot(h, m3_W[...]) + m3_b[...])
    logit = jnp.sum(h * p_W[...], axis=1) + p_b[0, 0]
    out_ref[...] = jax.nn.sigmoid(logit)


def kernel(user_ids, user_tags_idx, item_ids, content_features, user_table,
           tag_table, item_table, tp_W, tp_b, tp_g, tp_beta, uf_W, uf_b, uf_g,
           uf_beta, cf_W, cf_b, cf_g, cf_beta, if_W, if_b, if_g, if_beta,
           m1_W, m1_b, m2_W, m2_b, m3_W, m3_b, p_W, p_b):
    weights = [tp_W.T, tp_b.reshape(1, -1), tp_g.reshape(1, -1),
               tp_beta.reshape(1, -1), uf_W.T, uf_b.reshape(1, -1),
               uf_g.reshape(1, -1), uf_beta.reshape(1, -1), cf_W.T,
               cf_b.reshape(1, -1), cf_g.reshape(1, -1),
               cf_beta.reshape(1, -1), if_W.T, if_b.reshape(1, -1),
               if_g.reshape(1, -1), if_beta.reshape(1, -1), m1_W.T,
               m1_b.reshape(1, -1), m2_W.T, m2_b.reshape(1, -1), m3_W.T,
               m3_b.reshape(1, -1), p_W, p_b.reshape(1, -1)]

    BH = B // P
    BM = 2048
    row = pl.BlockSpec((BM, D), lambda i: (i, 0))
    full = lambda a: pl.BlockSpec(a.shape, lambda i: tuple(0 for _ in a.shape))
    dense = pl.pallas_call(
        _tc_body,
        grid=(BH // BM,),
        in_specs=[row, row, row, row] + [full(w) for w in weights],
        out_specs=pl.BlockSpec((BM,), lambda i: (i,)),
        out_shape=jax.ShapeDtypeStruct((BH,), jnp.float32),
    )

    outs = []
    for p in range(P):
        sl = slice(p * BH, (p + 1) * BH)
        uid, tsum, iid = _sc_gather(user_ids[sl], user_tags_idx[sl],
                                    item_ids[sl], user_table, tag_table,
                                    item_table)
        outs.append(dense(uid, tsum, iid, content_features[sl], *weights))
    return jnp.concatenate(outs)
